# Initial kernel scaffold; baseline (speedup 1.0000x reference)
import jax
import jax.numpy as jnp
from jax.experimental import pallas as pl


def kernel(word_inputs, feature_inputs, word_seq_lengths, char_inputs, char_seq_lengths, char_seq_recover, sw_inputs, sw_seqs_lengths, sw_seqs_recover, sw_fmasks, sw_bmasks, word_table, feat_table0):
    raise NotImplementedError("write your pallas kernel here")


# speedup vs baseline: 1.6611x; 1.6611x over previous
import functools

import jax
import jax.numpy as jnp
from jax import lax
from jax.experimental import pallas as pl
from jax.experimental.pallas import tpu as pltpu
from jax.experimental.pallas import tpu_sc as plsc

_B, _L = 1024, 200
_DW, _DF = 32, 16
_DO = _DW + _DF
_NTOK = _B * _L
_NC, _NS = 2, 16
_NW = _NC * _NS
_GSZ = 128
_RPU = 8
_UNIT = _RPU * _GSZ
_NUNITS = _NTOK // _UNIT
_MAXU = -(-_NUNITS // _NW)


def _make_sc_kernel():
    mesh = plsc.VectorSubcoreMesh(core_axis_name="c", subcore_axis_name="s")

    @functools.partial(
        pl.kernel,
        mesh=mesh,
        out_type=jax.ShapeDtypeStruct((_NTOK, _DO), jnp.float32),
        scratch_types=[
            pltpu.VMEM((_RPU, _GSZ), jnp.int32),
            pltpu.VMEM((_RPU, _GSZ), jnp.int32),
            pltpu.VMEM((_UNIT, _DW), jnp.float32),
            pltpu.VMEM((_UNIT, _DF), jnp.float32),
            pltpu.SemaphoreType.DMA,
        ],
        compiler_params=pltpu.CompilerParams(use_tc_tiling_on_sc=False),
    )
    def kern(widx_hbm, fidx_hbm, wtab_hbm, ftab_hbm, out_hbm,
             widx_v, fidx_v, wbuf, fbuf, gsem):
        wid = lax.axis_index("s") * _NC + lax.axis_index("c")
        nu = 6 + jnp.where(wid < _NUNITS - 6 * _NW, 1, 0)

        def body(u, carry):
            unit = wid + u * _NW
            grow0 = unit * _RPU
            tok0 = unit * _UNIT
            pltpu.sync_copy(widx_hbm.at[pl.ds(grow0, _RPU), :], widx_v)
            pltpu.sync_copy(fidx_hbm.at[pl.ds(grow0, _RPU), :], fidx_v)
            cps = []
            for j in range(_RPU):
                cps.append(pltpu.async_copy(
                    wtab_hbm.at[widx_v.at[j]],
                    wbuf.at[pl.ds(j * _GSZ, _GSZ), :], gsem))
                cps.append(pltpu.async_copy(
                    ftab_hbm.at[fidx_v.at[j]],
                    fbuf.at[pl.ds(j * _GSZ, _GSZ), :], gsem))
            for c in cps:
                c.wait()
            pltpu.sync_copy(wbuf, out_hbm.at[pl.ds(tok0, _UNIT), pl.ds(0, _DW)])
            pltpu.sync_copy(fbuf, out_hbm.at[pl.ds(tok0, _UNIT), pl.ds(_DW, _DF)])
            return carry

        lax.fori_loop(0, nu, body, 0)

    return kern


_SC_KERNEL = _make_sc_kernel()


def kernel(word_inputs, feature_inputs, word_seq_lengths, char_inputs,
           char_seq_lengths, char_seq_recover, sw_inputs, sw_seqs_lengths,
           sw_seqs_recover, sw_fmasks, sw_bmasks, word_table, feat_table0):
    widx = word_inputs.reshape(_NTOK // _GSZ, _GSZ)
    fidx = feature_inputs[0].reshape(_NTOK // _GSZ, _GSZ)
    out = _SC_KERNEL(widx, fidx, word_table, feat_table0)
    return out.reshape(_B, _L, _DO)


# speedup vs baseline: 1.6892x; 1.0169x over previous
import functools

import jax
import jax.numpy as jnp
from jax import lax
from jax.experimental import pallas as pl
from jax.experimental.pallas import tpu as pltpu
from jax.experimental.pallas import tpu_sc as plsc

_B, _L = 1024, 200
_DW, _DF = 32, 16
_DO = _DW + _DF
_NTOK = _B * _L
_NC, _NS = 2, 16
_NW = _NC * _NS
_GSZ = 128
_PERW = _NTOK // _NW
_NGW = _PERW // _GSZ
_RPC = 5
_CHUNK = _RPC * _GSZ
_NCH = _NGW // _RPC
_NBUF = 2


def _make_sc_kernel():
    mesh = plsc.VectorSubcoreMesh(core_axis_name="c", subcore_axis_name="s")

    @functools.partial(
        pl.kernel,
        mesh=mesh,
        out_type=jax.ShapeDtypeStruct((_NTOK, _DO), jnp.float32),
        scratch_types=[
            pltpu.VMEM((_NGW, _GSZ), jnp.int32),
            pltpu.VMEM((_NGW, _GSZ), jnp.int32),
            pltpu.VMEM((_NBUF, _CHUNK, _DW), jnp.float32),
            pltpu.VMEM((_NBUF, _CHUNK, _DF), jnp.float32),
            pltpu.SemaphoreType.DMA,
            pltpu.SemaphoreType.DMA,
        ],
        compiler_params=pltpu.CompilerParams(use_tc_tiling_on_sc=False),
    )
    def kern(widx_hbm, fidx_hbm, wtab_hbm, ftab_hbm, out_hbm,
             widx_v, fidx_v, wbuf, fbuf, gsem, wsem):
        wid = lax.axis_index("s") * _NC + lax.axis_index("c")
        row0 = wid * _NGW
        tok0w = wid * _PERW
        pltpu.sync_copy(widx_hbm.at[pl.ds(row0, _NGW), :], widx_v)
        pltpu.sync_copy(fidx_hbm.at[pl.ds(row0, _NGW), :], fidx_v)

        def wdst(c):
            return out_hbm.at[pl.ds(tok0w + c * _CHUNK, _CHUNK), pl.ds(0, _DW)]

        def fdst(c):
            return out_hbm.at[pl.ds(tok0w + c * _CHUNK, _CHUNK), pl.ds(_DW, _DF)]

        def body(ci, carry):
            slot = lax.rem(ci, _NBUF)

            @pl.when(ci >= _NBUF)
            def _():
                c2 = ci - _NBUF
                pltpu.make_async_copy(wbuf.at[slot], wdst(c2), wsem).wait()
                pltpu.make_async_copy(fbuf.at[slot], fdst(c2), wsem).wait()

            cps = []
            for j in range(_RPC):
                r = ci * _RPC + j
                cps.append(pltpu.async_copy(
                    wtab_hbm.at[widx_v.at[r]],
                    wbuf.at[slot, pl.ds(j * _GSZ, _GSZ), :], gsem))
                cps.append(pltpu.async_copy(
                    ftab_hbm.at[fidx_v.at[r]],
                    fbuf.at[slot, pl.ds(j * _GSZ, _GSZ), :], gsem))
            for c in cps:
                c.wait()

            pltpu.async_copy(wbuf.at[slot], wdst(ci), wsem)
            pltpu.async_copy(fbuf.at[slot], fdst(ci), wsem)
            return carry

        lax.fori_loop(0, _NCH, body, 0)

        for c in range(_NCH - _NBUF, _NCH):
            slot = c % _NBUF
            pltpu.make_async_copy(wbuf.at[slot], wdst(c), wsem).wait()
            pltpu.make_async_copy(fbuf.at[slot], fdst(c), wsem).wait()

    return kern


_SC_KERNEL = _make_sc_kernel()


def kernel(word_inputs, feature_inputs, word_seq_lengths, char_inputs,
           char_seq_lengths, char_seq_recover, sw_inputs, sw_seqs_lengths,
           sw_seqs_recover, sw_fmasks, sw_bmasks, word_table, feat_table0):
    widx = word_inputs.reshape(_NTOK // _GSZ, _GSZ)
    fidx = feature_inputs[0].reshape(_NTOK // _GSZ, _GSZ)
    out = _SC_KERNEL(widx, fidx, word_table, feat_table0)
    return out.reshape(_B, _L, _DO)


# speedup vs baseline: 1.6905x; 1.0008x over previous
import functools

import jax
import jax.numpy as jnp
from jax import lax
from jax.experimental import pallas as pl
from jax.experimental.pallas import tpu as pltpu
from jax.experimental.pallas import tpu_sc as plsc

_B, _L = 1024, 200
_DW, _DF = 32, 16
_DO = _DW + _DF
_NC, _NS = 2, 16
_NW = _NC * _NS
_SPW = _B // _NW
_SPC = 4
_NCH = _SPW // _SPC
_NBUF = 2
_SPLITS = ((0, 128), (128, _L - 128))


def _make_sc_kernel():
    mesh = plsc.VectorSubcoreMesh(core_axis_name="c", subcore_axis_name="s")

    @functools.partial(
        pl.kernel,
        mesh=mesh,
        out_type=jax.ShapeDtypeStruct((_B, _L, _DO), jnp.float32),
        scratch_types=[
            pltpu.VMEM((_SPW, _L), jnp.int32),
            pltpu.VMEM((_SPW, _L), jnp.int32),
            pltpu.VMEM((_NBUF, _SPC, _L, _DW), jnp.float32),
            pltpu.VMEM((_NBUF, _SPC, _L, _DF), jnp.float32),
            pltpu.SemaphoreType.DMA,
            pltpu.SemaphoreType.DMA,
        ],
        compiler_params=pltpu.CompilerParams(use_tc_tiling_on_sc=False),
    )
    def kern(widx_hbm, fidx_hbm, wtab_hbm, ftab_hbm, out_hbm,
             widx_v, fidx_v, wbuf, fbuf, gsem, wsem):
        wid = lax.axis_index("s") * _NC + lax.axis_index("c")
        s0w = wid * _SPW
        pltpu.sync_copy(widx_hbm.at[pl.ds(s0w, _SPW), :], widx_v)
        pltpu.sync_copy(fidx_hbm.at[0, pl.ds(s0w, _SPW), :], fidx_v)

        def wdst(c):
            return out_hbm.at[pl.ds(s0w + c * _SPC, _SPC), :, pl.ds(0, _DW)]

        def fdst(c):
            return out_hbm.at[pl.ds(s0w + c * _SPC, _SPC), :, pl.ds(_DW, _DF)]

        def body(ci, carry):
            slot = lax.rem(ci, _NBUF)

            @pl.when(ci >= _NBUF)
            def _():
                c2 = ci - _NBUF
                pltpu.make_async_copy(wbuf.at[slot], wdst(c2), wsem).wait()
                pltpu.make_async_copy(fbuf.at[slot], fdst(c2), wsem).wait()

            cps = []
            for si in range(_SPC):
                r = ci * _SPC + si
                for c0, nc in _SPLITS:
                    cps.append(pltpu.async_copy(
                        wtab_hbm.at[widx_v.at[r, pl.ds(c0, nc)]],
                        wbuf.at[slot, si, pl.ds(c0, nc), :], gsem))
                    cps.append(pltpu.async_copy(
                        ftab_hbm.at[fidx_v.at[r, pl.ds(c0, nc)]],
                        fbuf.at[slot, si, pl.ds(c0, nc), :], gsem))
            for c in cps:
                c.wait()

            pltpu.async_copy(wbuf.at[slot], wdst(ci), wsem)
            pltpu.async_copy(fbuf.at[slot], fdst(ci), wsem)
            return carry

        lax.fori_loop(0, _NCH, body, 0)

        for c in range(_NCH - _NBUF, _NCH):
            slot = c % _NBUF
            pltpu.make_async_copy(wbuf.at[slot], wdst(c), wsem).wait()
            pltpu.make_async_copy(fbuf.at[slot], fdst(c), wsem).wait()

    return kern


_SC_KERNEL = _make_sc_kernel()


def kernel(word_inputs, feature_inputs, word_seq_lengths, char_inputs,
           char_seq_lengths, char_seq_recover, sw_inputs, sw_seqs_lengths,
           sw_seqs_recover, sw_fmasks, sw_bmasks, word_table, feat_table0):
    return _SC_KERNEL(word_inputs, feature_inputs, word_table, feat_table0)
